# Initial kernel scaffold; baseline (speedup 1.0000x reference)
import jax
import jax.numpy as jnp
from jax.experimental import pallas as pl


def kernel(x, norm1_g, norm1_b, qkv_w, qkv_b, rpb_table, proj_w, proj_b, norm2_g, norm2_b, gate_w, gate_b, fc1_w, fc1_b, fc2_w, fc2_b, rel_pos_index):
    raise NotImplementedError("write your pallas kernel here")


# speedup vs baseline: 2.7879x; 2.7879x over previous
import functools
import jax
import jax.numpy as jnp
from jax.experimental import pallas as pl
from jax.experimental.pallas import tpu as pltpu

B, H, W, C = 8, 32, 32, 768
WS = 8
NH = 12
E = 8
HID = 3072
DH = C // NH
N = WS * WS
NB = H // WS
TOK = B * H * W
T = 128
G = TOK // T + E
P = G * T
NTAB = (2 * WS - 1) ** 2


def _bias_kernel(rpb_t_ref, idx_ref, out_ref):
    idx = idx_ref[0, :]
    iot = jax.lax.broadcasted_iota(jnp.int32, (NTAB, N * N), 0)
    onehot = (iot == idx[None, :]).astype(jnp.float32)
    out_ref[...] = jnp.dot(rpb_t_ref[...], onehot,
                           preferred_element_type=jnp.float32)


def _attn_kernel(x_ref, n1g_ref, n1b_ref, qkvw_ref, qkvb_ref, bias_ref,
                 projw_ref, projb_ref, n2g_ref, n2b_ref, gw_ref, gb_ref,
                 out_ref, eid_ref, attn_scratch):
    x = x_ref[0, 0].reshape(WS * W, C)
    m = jnp.mean(x, axis=-1, keepdims=True)
    v = jnp.mean((x - m) ** 2, axis=-1, keepdims=True)
    h = (x - m) * jax.lax.rsqrt(v + 1e-5) * n1g_ref[0, :] + n1b_ref[0, :]
    qkv = jax.lax.dot_general(h, qkvw_ref[...], (((1,), (1,)), ((), ())),
                              preferred_element_type=jnp.float32)
    qkv = qkv + qkvb_ref[0, :]
    qkv3 = qkv.reshape(WS, W, 3 * C)
    scale = DH ** -0.5
    for wc in range(W // WS):
        blk = qkv3[:, wc * WS:(wc + 1) * WS, :].reshape(N, 3 * C)
        for hh in range(NH):
            q = blk[:, hh * DH:(hh + 1) * DH]
            k = blk[:, C + hh * DH:C + (hh + 1) * DH]
            vv = blk[:, 2 * C + hh * DH:2 * C + (hh + 1) * DH]
            s = jax.lax.dot_general(q * scale, k, (((1,), (1,)), ((), ())),
                                    preferred_element_type=jnp.float32)
            s = s + bias_ref[hh].reshape(N, N)
            s = s - jnp.max(s, axis=-1, keepdims=True)
            p = jnp.exp(s)
            p = p / jnp.sum(p, axis=-1, keepdims=True)
            o = jnp.dot(p, vv, preferred_element_type=jnp.float32)
            attn_scratch[:, wc * WS:(wc + 1) * WS, hh * DH:(hh + 1) * DH] = (
                o.reshape(WS, WS, DH))
    ao = attn_scratch[...].reshape(WS * W, C)
    proj = jax.lax.dot_general(ao, projw_ref[...], (((1,), (1,)), ((), ())),
                               preferred_element_type=jnp.float32)
    x2 = x + proj + projb_ref[0, :]
    out_ref[0, 0] = x2.reshape(WS, W, C)
    m2 = jnp.mean(x2, axis=-1, keepdims=True)
    v2 = jnp.mean((x2 - m2) ** 2, axis=-1, keepdims=True)
    h2 = (x2 - m2) * jax.lax.rsqrt(v2 + 1e-5) * n2g_ref[0, :] + n2b_ref[0, :]
    logits = jax.lax.dot_general(h2, gw_ref[...], (((1,), (1,)), ((), ())),
                                 preferred_element_type=jnp.float32)
    logits = logits + gb_ref[0, :]
    best_v = logits[:, 0]
    best_i = jnp.zeros((WS * W,), jnp.int32)
    for e in range(1, E):
        le = logits[:, e]
        upd = le > best_v
        best_v = jnp.where(upd, le, best_v)
        best_i = jnp.where(upd, e, best_i)
    eid_ref[0, 0, 0] = best_i


def _moe_kernel(be_ref, xs_ref, n2g_ref, n2b_ref, w1_ref, b1_ref,
                w2_ref, b2_ref, out_ref):
    x = xs_ref[...]
    m = jnp.mean(x, axis=-1, keepdims=True)
    v = jnp.mean((x - m) ** 2, axis=-1, keepdims=True)
    h2 = (x - m) * jax.lax.rsqrt(v + 1e-5) * n2g_ref[0, :] + n2b_ref[0, :]
    he = jax.lax.dot_general(h2, w1_ref[0], (((1,), (1,)), ((), ())),
                             preferred_element_type=jnp.float32)
    he = he + b1_ref[0, 0]
    he = 0.5 * he * (1.0 + jax.lax.erf(he * (2.0 ** -0.5)))
    o = jax.lax.dot_general(he, w2_ref[0], (((1,), (1,)), ((), ())),
                            preferred_element_type=jnp.float32)
    out_ref[...] = x + o + b2_ref[0, 0]


def _full(shape):
    return pl.BlockSpec(shape, lambda *_: tuple(0 for _ in shape))


def kernel(x, norm1_g, norm1_b, qkv_w, qkv_b, rpb_table, proj_w, proj_b,
           norm2_g, norm2_b, gate_w, gate_b, fc1_w, fc1_b, fc2_w, fc2_b,
           rel_pos_index):
    f32 = jnp.float32
    bias = pl.pallas_call(
        _bias_kernel,
        grid=(1,),
        in_specs=[_full((NH, NTAB)), _full((1, N * N))],
        out_specs=_full((NH, N * N)),
        out_shape=jax.ShapeDtypeStruct((NH, N * N), f32),
    )(rpb_table.T, rel_pos_index.reshape(1, N * N))
    bias = bias.reshape(NH, N, N)

    xb = x.reshape(B, NB, WS, W, C)
    r1 = lambda b, r: (b, r, 0, 0, 0)
    x2b, eid = pl.pallas_call(
        _attn_kernel,
        grid=(B, NB),
        in_specs=[
            pl.BlockSpec((1, 1, WS, W, C), r1),
            _full((1, C)), _full((1, C)),
            _full((3 * C, C)), _full((1, 3 * C)),
            _full((NH, N, N)),
            _full((C, C)), _full((1, C)),
            _full((1, C)), _full((1, C)),
            _full((E, C)), _full((1, E)),
        ],
        out_specs=[
            pl.BlockSpec((1, 1, WS, W, C), r1),
            pl.BlockSpec((1, 1, 1, WS * W), lambda b, r: (b, r, 0, 0)),
        ],
        out_shape=[
            jax.ShapeDtypeStruct((B, NB, WS, W, C), f32),
            jax.ShapeDtypeStruct((B, NB, 1, WS * W), jnp.int32),
        ],
        scratch_shapes=[pltpu.VMEM((WS, W, C), f32)],
    )(xb, norm1_g.reshape(1, C), norm1_b.reshape(1, C), qkv_w,
      qkv_b.reshape(1, 3 * C), bias, proj_w, proj_b.reshape(1, C),
      norm2_g.reshape(1, C), norm2_b.reshape(1, C), gate_w,
      gate_b.reshape(1, E))
    x2 = x2b.reshape(TOK, C)
    e_tok = eid.reshape(TOK)

    counts = jnp.bincount(e_tok, length=E)
    padded = ((counts + T - 1) // T) * T
    off = jnp.cumsum(padded) - padded
    csum_excl = jnp.cumsum(counts) - counts
    order = jnp.argsort(e_tok, stable=True)
    inv = jnp.zeros((TOK,), jnp.int32).at[order].set(
        jnp.arange(TOK, dtype=jnp.int32))
    dst = (off[e_tok] + (inv - csum_excl[e_tok])).astype(jnp.int32)
    x_sorted = jnp.zeros((P, C), f32).at[dst].set(x2)
    blk_expert = jnp.searchsorted(
        off[1:], jnp.arange(G, dtype=jnp.int32) * T, side='right'
    ).astype(jnp.int32)

    y_sorted = pl.pallas_call(
        _moe_kernel,
        grid_spec=pltpu.PrefetchScalarGridSpec(
            num_scalar_prefetch=1,
            grid=(G,),
            in_specs=[
                pl.BlockSpec((T, C), lambda g, s: (g, 0)),
                _full((1, C)), _full((1, C)),
                pl.BlockSpec((1, HID, C), lambda g, s: (s[g], 0, 0)),
                pl.BlockSpec((1, 1, HID), lambda g, s: (s[g], 0, 0)),
                pl.BlockSpec((1, C, HID), lambda g, s: (s[g], 0, 0)),
                pl.BlockSpec((1, 1, C), lambda g, s: (s[g], 0, 0)),
            ],
            out_specs=pl.BlockSpec((T, C), lambda g, s: (g, 0)),
        ),
        out_shape=jax.ShapeDtypeStruct((P, C), f32),
    )(blk_expert, x_sorted, norm2_g.reshape(1, C), norm2_b.reshape(1, C),
      fc1_w, fc1_b.reshape(E, 1, HID), fc2_w, fc2_b.reshape(E, 1, C))

    out = y_sorted[dst].reshape(B, H * W, C)
    return (out, jnp.float32(0.0))


# speedup vs baseline: 2.8316x; 1.0157x over previous
import functools
import jax
import jax.numpy as jnp
from jax import lax
from jax.experimental import pallas as pl
from jax.experimental.pallas import tpu as pltpu
from jax.experimental.pallas import tpu_sc as plsc

B, H, W, C = 8, 32, 32, 768
WS = 8
NH = 12
E = 8
HID = 3072
DH = C // NH
N = WS * WS
NB = H // WS
TOK = B * H * W
T = 128
NTAB = (2 * WS - 1) ** 2
NC = 2
NS = 16
L = 16
HALF = TOK // NC
HALF_P = HALF + E * T
P = NC * HALF_P
G = P // T
TPW = TOK // (NC * NS)


def _bias_kernel(rpb_t_ref, idx_ref, out_ref):
    idx = idx_ref[0, :]
    iot = jax.lax.broadcasted_iota(jnp.int32, (NTAB, N * N), 0)
    onehot = (iot == idx[None, :]).astype(jnp.float32)
    out_ref[...] = jnp.dot(rpb_t_ref[...], onehot,
                           preferred_element_type=jnp.float32,
                           precision=jax.lax.Precision.HIGHEST)


def _attn_kernel(x_ref, n1g_ref, n1b_ref, qkvw_ref, qkvb_ref, bias_ref,
                 projw_ref, projb_ref, n2g_ref, n2b_ref, gw_ref, gb_ref,
                 out_ref, eid_ref, attn_scratch):
    x = x_ref[0, 0].reshape(WS * W, C)
    m = jnp.mean(x, axis=-1, keepdims=True)
    v = jnp.mean((x - m) ** 2, axis=-1, keepdims=True)
    h = (x - m) / jnp.sqrt(v + 1e-5) * n1g_ref[0, :] + n1b_ref[0, :]
    qkv = jax.lax.dot_general(h, qkvw_ref[...], (((1,), (1,)), ((), ())),
                              preferred_element_type=jnp.float32)
    qkv = qkv + qkvb_ref[0, :]
    qkv3 = qkv.reshape(WS, W, 3 * C)
    scale = DH ** -0.5
    for wc in range(W // WS):
        blk = qkv3[:, wc * WS:(wc + 1) * WS, :].reshape(N, 3 * C)
        for hh in range(NH):
            q = blk[:, hh * DH:(hh + 1) * DH]
            k = blk[:, C + hh * DH:C + (hh + 1) * DH]
            vv = blk[:, 2 * C + hh * DH:2 * C + (hh + 1) * DH]
            s = jax.lax.dot_general(q * scale, k, (((1,), (1,)), ((), ())),
                                    preferred_element_type=jnp.float32)
            s = s + bias_ref[hh].reshape(N, N)
            s = s - jnp.max(s, axis=-1, keepdims=True)
            p = jnp.exp(s)
            p = p / jnp.sum(p, axis=-1, keepdims=True)
            o = jnp.dot(p, vv, preferred_element_type=jnp.float32)
            attn_scratch[:, wc * WS:(wc + 1) * WS, hh * DH:(hh + 1) * DH] = (
                o.reshape(WS, WS, DH))
    ao = attn_scratch[...].reshape(WS * W, C)
    proj = jax.lax.dot_general(ao, projw_ref[...], (((1,), (1,)), ((), ())),
                               preferred_element_type=jnp.float32)
    x2 = x + (proj + projb_ref[0, :])
    out_ref[0, 0] = x2.reshape(WS, W, C)
    m2 = jnp.mean(x2, axis=-1, keepdims=True)
    v2 = jnp.mean((x2 - m2) ** 2, axis=-1, keepdims=True)
    h2 = (x2 - m2) / jnp.sqrt(v2 + 1e-5) * n2g_ref[0, :] + n2b_ref[0, :]
    logits = jax.lax.dot_general(h2, gw_ref[...], (((1,), (1,)), ((), ())),
                                 preferred_element_type=jnp.float32)
    logits = logits + gb_ref[0, :]
    best_v = logits[:, 0]
    best_i = jnp.zeros((WS * W,), jnp.int32)
    for e in range(1, E):
        le = logits[:, e]
        upd = le > best_v
        best_v = jnp.where(upd, le, best_v)
        best_i = jnp.where(upd, e, best_i)
    eid_ref[0, 0, 0] = best_i


def _moe_kernel(be_ref, xs_ref, n2g_ref, n2b_ref, w1_ref, b1_ref,
                w2_ref, b2_ref, out_ref):
    x = xs_ref[...]
    m = jnp.mean(x, axis=-1, keepdims=True)
    v = jnp.mean((x - m) ** 2, axis=-1, keepdims=True)
    h2 = (x - m) / jnp.sqrt(v + 1e-5) * n2g_ref[0, :] + n2b_ref[0, :]
    he = jax.lax.dot_general(h2, w1_ref[0], (((1,), (1,)), ((), ())),
                             preferred_element_type=jnp.float32)
    he = he + b1_ref[0, 0]
    he = 0.5 * he * (1.0 + jax.lax.erf(he * (2.0 ** -0.5)))
    o = jax.lax.dot_general(he, w2_ref[0], (((1,), (1,)), ((), ())),
                            preferred_element_type=jnp.float32)
    out_ref[...] = x + o + b2_ref[0, 0]


def _full(shape):
    return pl.BlockSpec(shape, lambda *_: tuple(0 for _ in shape))


def _route_kernel(eid_hbm, x2_hbm, xs_hbm, dst_hbm, be_hbm,
                  eid_v, xrow_v, hist_v, hist_all_v, cur_v, off_v, dst_v,
                  be_v, hist_sh, sem):
    c = lax.axis_index("c")
    s = lax.axis_index("s")
    tok0 = c * HALF + s * TPW
    pltpu.sync_copy(eid_hbm.at[pl.ds(tok0, TPW)], eid_v)
    iot = lax.broadcasted_iota(jnp.int32, (L,), 0)
    hist = jnp.zeros((L,), jnp.int32)
    for j in range(TPW // L):
        v = eid_v[pl.ds(j * L, L)]
        for e in range(E):
            pc = jnp.sum(jnp.where(v == e, 1, 0))
            hist = jnp.where(iot == e + 1, hist + pc, hist)
    hist_v[...] = hist
    pltpu.sync_copy(hist_v, hist_sh.at[pl.ds(s * L, L)])
    plsc.subcore_barrier()
    pltpu.sync_copy(hist_sh, hist_all_v)
    total = jnp.zeros((L,), jnp.int32)
    prefix = jnp.zeros((L,), jnp.int32)
    for t in range(NS):
        row = hist_all_v[pl.ds(t * L, L)]
        total = total + row
        prefix = jnp.where(jnp.full((L,), t, jnp.int32) < s,
                           prefix + row, prefix)
    padded = jnp.where(iot == 0, 0, ((total + (T - 1)) >> 7) << 7)
    csum = plsc.cumsum(padded)
    off_local = csum - padded
    off_v[...] = off_local
    cur_v[...] = off_local + prefix + c * HALF_P
    @pl.when(s == 0)
    def _():
        for ch in range(3):
            g = lax.broadcasted_iota(jnp.int32, (L,), 0) + ch * L
            slot = g * T
            be = jnp.zeros((L,), jnp.int32)
            for e in range(1, E):
                off_e = plsc.load_gather(off_v,
                                         [jnp.full((L,), e + 1, jnp.int32)])
                be = be + jnp.where(slot >= off_e, 1, 0)
            be_v[pl.ds(ch * L, L)] = be
        pltpu.sync_copy(be_v.at[pl.ds(0, HALF_P // T)],
                        be_hbm.at[pl.ds(c * (HALF_P // T), HALF_P // T)])
    for j in range(TPW // L):
        v = eid_v[pl.ds(j * L, L)]
        dst = jnp.zeros((L,), jnp.int32)
        hist_c = jnp.zeros((L,), jnp.int32)
        for e in range(E):
            cur_e = plsc.load_gather(cur_v, [jnp.full((L,), e + 1, jnp.int32)])
            mask = v == e
            mc = plsc.cumsum(jnp.where(mask, 1, 0))
            rank = mc - 1
            dst = jnp.where(mask, cur_e + rank, dst)
            pc = jnp.sum(jnp.where(mask, 1, 0))
            hist_c = jnp.where(iot == e + 1, hist_c + pc, hist_c)
        cur_v[...] = cur_v[...] + hist_c
        dst_v[pl.ds(j * L, L)] = dst
        pltpu.sync_copy(x2_hbm.at[pl.ds(tok0 + j * L, L)], xrow_v)
        pltpu.async_copy(xrow_v, xs_hbm.at[dst], sem).wait()
    pltpu.sync_copy(dst_v, dst_hbm.at[pl.ds(tok0, TPW)])


def _unsort_kernel(ys_hbm, dst_hbm, out_hbm, dst_v, yrow_v, sem):
    c = lax.axis_index("c")
    s = lax.axis_index("s")
    tok0 = (s * NC + c) * TPW
    pltpu.sync_copy(dst_hbm.at[pl.ds(tok0, TPW)], dst_v)
    for j in range(TPW // L):
        idx = dst_v[pl.ds(j * L, L)]
        pltpu.async_copy(ys_hbm.at[idx], yrow_v, sem).wait()
        pltpu.sync_copy(yrow_v, out_hbm.at[pl.ds(tok0 + j * L, L)])


def kernel(x, norm1_g, norm1_b, qkv_w, qkv_b, rpb_table, proj_w, proj_b,
           norm2_g, norm2_b, gate_w, gate_b, fc1_w, fc1_b, fc2_w, fc2_b,
           rel_pos_index):
    f32 = jnp.float32
    bias = pl.pallas_call(
        _bias_kernel,
        grid=(1,),
        in_specs=[_full((NH, NTAB)), _full((1, N * N))],
        out_specs=_full((NH, N * N)),
        out_shape=jax.ShapeDtypeStruct((NH, N * N), f32),
    )(rpb_table.T, rel_pos_index.reshape(1, N * N))
    bias = bias.reshape(NH, N, N)

    xb = x.reshape(B, NB, WS, W, C)
    r1 = lambda b, r: (b, r, 0, 0, 0)
    x2b, eid = pl.pallas_call(
        _attn_kernel,
        grid=(B, NB),
        in_specs=[
            pl.BlockSpec((1, 1, WS, W, C), r1),
            _full((1, C)), _full((1, C)),
            _full((3 * C, C)), _full((1, 3 * C)),
            _full((NH, N, N)),
            _full((C, C)), _full((1, C)),
            _full((1, C)), _full((1, C)),
            _full((E, C)), _full((1, E)),
        ],
        out_specs=[
            pl.BlockSpec((1, 1, WS, W, C), r1),
            pl.BlockSpec((1, 1, 1, WS * W), lambda b, r: (b, r, 0, 0)),
        ],
        out_shape=[
            jax.ShapeDtypeStruct((B, NB, WS, W, C), f32),
            jax.ShapeDtypeStruct((B, NB, 1, WS * W), jnp.int32),
        ],
        scratch_shapes=[pltpu.VMEM((WS, W, C), f32)],
    )(xb, norm1_g.reshape(1, C), norm1_b.reshape(1, C), qkv_w,
      qkv_b.reshape(1, 3 * C), bias, proj_w, proj_b.reshape(1, C),
      norm2_g.reshape(1, C), norm2_b.reshape(1, C), gate_w,
      gate_b.reshape(1, E))
    x2 = x2b.reshape(TOK, C)
    e_tok = eid.reshape(TOK)

    mesh = plsc.VectorSubcoreMesh(core_axis_name="c", subcore_axis_name="s")
    x_sorted, dst, blk_expert = pl.kernel(
        _route_kernel,
        mesh=mesh,
        compiler_params=pltpu.CompilerParams(needs_layout_passes=False),
        out_type=[
            jax.ShapeDtypeStruct((P, C), f32),
            jax.ShapeDtypeStruct((TOK,), jnp.int32),
            jax.ShapeDtypeStruct((G,), jnp.int32),
        ],
        scratch_types=[
            pltpu.VMEM((TPW,), jnp.int32),
            pltpu.VMEM((L, C), f32),
            pltpu.VMEM((L,), jnp.int32),
            pltpu.VMEM((NS * L,), jnp.int32),
            pltpu.VMEM((L,), jnp.int32),
            pltpu.VMEM((L,), jnp.int32),
            pltpu.VMEM((TPW,), jnp.int32),
            pltpu.VMEM((3 * L,), jnp.int32),
            pltpu.VMEM_SHARED((NS * L,), jnp.int32),
            pltpu.SemaphoreType.DMA,
        ],
    )(e_tok, x2)

    y_sorted = pl.pallas_call(
        _moe_kernel,
        grid_spec=pltpu.PrefetchScalarGridSpec(
            num_scalar_prefetch=1,
            grid=(G,),
            in_specs=[
                pl.BlockSpec((T, C), lambda g, s: (g, 0)),
                _full((1, C)), _full((1, C)),
                pl.BlockSpec((1, HID, C), lambda g, s: (s[g], 0, 0)),
                pl.BlockSpec((1, 1, HID), lambda g, s: (s[g], 0, 0)),
                pl.BlockSpec((1, C, HID), lambda g, s: (s[g], 0, 0)),
                pl.BlockSpec((1, 1, C), lambda g, s: (s[g], 0, 0)),
            ],
            out_specs=pl.BlockSpec((T, C), lambda g, s: (g, 0)),
        ),
        out_shape=jax.ShapeDtypeStruct((P, C), f32),
    )(blk_expert, x_sorted, norm2_g.reshape(1, C), norm2_b.reshape(1, C),
      fc1_w, fc1_b.reshape(E, 1, HID), fc2_w, fc2_b.reshape(E, 1, C))

    out_flat = pl.kernel(
        _unsort_kernel,
        mesh=mesh,
        compiler_params=pltpu.CompilerParams(needs_layout_passes=False),
        out_type=jax.ShapeDtypeStruct((TOK, C), f32),
        scratch_types=[
            pltpu.VMEM((TPW,), jnp.int32),
            pltpu.VMEM((L, C), f32),
            pltpu.SemaphoreType.DMA,
        ],
    )(y_sorted, dst)
    out = out_flat.reshape(B, H * W, C)
    return (out, jnp.float32(0.0))


# speedup vs baseline: 4.6445x; 1.6402x over previous
import functools
import jax
import jax.numpy as jnp
from jax import lax
from jax.experimental import pallas as pl
from jax.experimental.pallas import tpu as pltpu
from jax.experimental.pallas import tpu_sc as plsc

B, H, W, C = 8, 32, 32, 768
WS = 8
NH = 12
E = 8
HID = 3072
DH = C // NH
N = WS * WS
NB = H // WS
TOK = B * H * W
T = 128
NTAB = (2 * WS - 1) ** 2
NC = 2
NS = 16
L = 16
HALF = TOK // NC
HALF_P = HALF + E * T
P = NC * HALF_P
G = P // T
TPW = TOK // (NC * NS)


def _bias_kernel(rpb_t_ref, idx_ref, out_ref):
    idx = idx_ref[0, :]
    iot = jax.lax.broadcasted_iota(jnp.int32, (NTAB, N * N), 0)
    onehot = (iot == idx[None, :]).astype(jnp.float32)
    out_ref[...] = jnp.dot(rpb_t_ref[...], onehot,
                           preferred_element_type=jnp.float32,
                           precision=jax.lax.Precision.HIGHEST)


def _attn_kernel(x_ref, n1g_ref, n1b_ref, qkvw_ref, qkvb_ref, bias_ref,
                 projw_ref, projb_ref, n2g_ref, n2b_ref, gw_ref, gb_ref,
                 out_ref, eid_ref, attn_scratch, sc_scratch):
    x = x_ref[0, 0].reshape(WS * W, C)
    m = jnp.mean(x, axis=-1, keepdims=True)
    v = jnp.mean((x - m) ** 2, axis=-1, keepdims=True)
    h = (x - m) / jnp.sqrt(v + 1e-5) * n1g_ref[0, :] + n1b_ref[0, :]
    qkv = jax.lax.dot_general(h, qkvw_ref[...], (((1,), (1,)), ((), ())),
                              preferred_element_type=jnp.float32)
    qkv = qkv + qkvb_ref[0, :]
    qkv3 = qkv.reshape(WS, W, 3 * C)
    scale = DH ** -0.5
    nw = W // WS
    for wc in range(nw):
        blk = qkv3[:, wc * WS:(wc + 1) * WS, :].reshape(N, 3 * C)
        for hh in range(NH):
            q = blk[:, hh * DH:(hh + 1) * DH]
            k = blk[:, C + hh * DH:C + (hh + 1) * DH]
            s = jax.lax.dot_general(q * scale, k, (((1,), (1,)), ((), ())),
                                    preferred_element_type=jnp.float32)
            sc_scratch[(wc * NH + hh) * N:(wc * NH + hh + 1) * N, :] = s
    bias_all = bias_ref[...].reshape(NH * N, N)
    bias_t = jnp.concatenate([bias_all] * nw, axis=0)
    s_all = sc_scratch[...] + bias_t
    s_all = s_all - jnp.max(s_all, axis=-1, keepdims=True)
    p_all = jnp.exp(s_all)
    p_all = p_all / jnp.sum(p_all, axis=-1, keepdims=True)
    sc_scratch[...] = p_all
    for wc in range(nw):
        blk = qkv3[:, wc * WS:(wc + 1) * WS, :].reshape(N, 3 * C)
        for hh in range(NH):
            vv = blk[:, 2 * C + hh * DH:2 * C + (hh + 1) * DH]
            p = sc_scratch[(wc * NH + hh) * N:(wc * NH + hh + 1) * N, :]
            o = jnp.dot(p, vv, preferred_element_type=jnp.float32)
            attn_scratch[:, wc * WS:(wc + 1) * WS, hh * DH:(hh + 1) * DH] = (
                o.reshape(WS, WS, DH))
    ao = attn_scratch[...].reshape(WS * W, C)
    proj = jax.lax.dot_general(ao, projw_ref[...], (((1,), (1,)), ((), ())),
                               preferred_element_type=jnp.float32)
    x2 = x + (proj + projb_ref[0, :])
    out_ref[0, 0] = x2.reshape(WS, W, C)
    m2 = jnp.mean(x2, axis=-1, keepdims=True)
    v2 = jnp.mean((x2 - m2) ** 2, axis=-1, keepdims=True)
    h2 = (x2 - m2) / jnp.sqrt(v2 + 1e-5) * n2g_ref[0, :] + n2b_ref[0, :]
    logits = jax.lax.dot_general(h2, gw_ref[...], (((1,), (1,)), ((), ())),
                                 preferred_element_type=jnp.float32)
    logits = logits + gb_ref[0, :]
    best_v = logits[:, 0]
    best_i = jnp.zeros((WS * W,), jnp.int32)
    for e in range(1, E):
        le = logits[:, e]
        upd = le > best_v
        best_v = jnp.where(upd, le, best_v)
        best_i = jnp.where(upd, e, best_i)
    eid_ref[0, 0, 0] = best_i


def _moe_kernel(be_ref, xs_ref, n2g_ref, n2b_ref, w1_ref, b1_ref,
                w2_ref, b2_ref, out_ref):
    x = xs_ref[...]
    m = jnp.mean(x, axis=-1, keepdims=True)
    v = jnp.mean((x - m) ** 2, axis=-1, keepdims=True)
    h2 = (x - m) / jnp.sqrt(v + 1e-5) * n2g_ref[0, :] + n2b_ref[0, :]
    he = jax.lax.dot_general(h2, w1_ref[0], (((1,), (1,)), ((), ())),
                             preferred_element_type=jnp.float32)
    he = he + b1_ref[0, 0]
    he = 0.5 * he * (1.0 + jax.lax.erf(he * (2.0 ** -0.5)))
    o = jax.lax.dot_general(he, w2_ref[0], (((1,), (1,)), ((), ())),
                            preferred_element_type=jnp.float32)
    out_ref[...] = x + o + b2_ref[0, 0]


def _full(shape):
    return pl.BlockSpec(shape, lambda *_: tuple(0 for _ in shape))


def _route_kernel(eid_hbm, x2_hbm, xs_hbm, dst_hbm, be_hbm,
                  eid_v, xrow_v, hist_v, hist_all_v, cur_v, off_v, dst_v,
                  be_v, hist_sh, sem):
    c = lax.axis_index("c")
    s = lax.axis_index("s")
    tok0 = c * HALF + s * TPW
    pltpu.sync_copy(eid_hbm.at[pl.ds(tok0, TPW)], eid_v)
    iot = lax.broadcasted_iota(jnp.int32, (L,), 0)
    hist = jnp.zeros((L,), jnp.int32)
    for j in range(TPW // L):
        v = eid_v[pl.ds(j * L, L)]
        for e in range(E):
            pc = jnp.sum(jnp.where(v == e, 1, 0))
            hist = jnp.where(iot == e + 1, hist + pc, hist)
    hist_v[...] = hist
    pltpu.sync_copy(hist_v, hist_sh.at[pl.ds(s * L, L)])
    plsc.subcore_barrier()
    pltpu.sync_copy(hist_sh, hist_all_v)
    total = jnp.zeros((L,), jnp.int32)
    prefix = jnp.zeros((L,), jnp.int32)
    for t in range(NS):
        row = hist_all_v[pl.ds(t * L, L)]
        total = total + row
        prefix = jnp.where(jnp.full((L,), t, jnp.int32) < s,
                           prefix + row, prefix)
    padded = jnp.where(iot == 0, 0, ((total + (T - 1)) >> 7) << 7)
    csum = plsc.cumsum(padded)
    off_local = csum - padded
    off_v[...] = off_local
    cur_v[...] = off_local + prefix + c * HALF_P
    @pl.when(s == 0)
    def _():
        for ch in range(3):
            g = lax.broadcasted_iota(jnp.int32, (L,), 0) + ch * L
            slot = g * T
            be = jnp.zeros((L,), jnp.int32)
            for e in range(1, E):
                off_e = plsc.load_gather(off_v,
                                         [jnp.full((L,), e + 1, jnp.int32)])
                be = be + jnp.where(slot >= off_e, 1, 0)
            be_v[pl.ds(ch * L, L)] = be
        pltpu.sync_copy(be_v.at[pl.ds(0, HALF_P // T)],
                        be_hbm.at[pl.ds(c * (HALF_P // T), HALF_P // T)])
    for j in range(TPW // L):
        v = eid_v[pl.ds(j * L, L)]
        dst = jnp.zeros((L,), jnp.int32)
        hist_c = jnp.zeros((L,), jnp.int32)
        for e in range(E):
            cur_e = plsc.load_gather(cur_v, [jnp.full((L,), e + 1, jnp.int32)])
            mask = v == e
            mc = plsc.cumsum(jnp.where(mask, 1, 0))
            rank = mc - 1
            dst = jnp.where(mask, cur_e + rank, dst)
            pc = jnp.sum(jnp.where(mask, 1, 0))
            hist_c = jnp.where(iot == e + 1, hist_c + pc, hist_c)
        cur_v[...] = cur_v[...] + hist_c
        dst_v[pl.ds(j * L, L)] = dst
        pltpu.sync_copy(x2_hbm.at[pl.ds(tok0 + j * L, L)], xrow_v)
        pltpu.async_copy(xrow_v, xs_hbm.at[dst], sem).wait()
    pltpu.sync_copy(dst_v, dst_hbm.at[pl.ds(tok0, TPW)])


def _unsort_kernel(ys_hbm, dst_hbm, out_hbm, dst_v, yrow_v, sem):
    c = lax.axis_index("c")
    s = lax.axis_index("s")
    tok0 = (s * NC + c) * TPW
    pltpu.sync_copy(dst_hbm.at[pl.ds(tok0, TPW)], dst_v)
    for j in range(TPW // L):
        idx = dst_v[pl.ds(j * L, L)]
        pltpu.async_copy(ys_hbm.at[idx], yrow_v, sem).wait()
        pltpu.sync_copy(yrow_v, out_hbm.at[pl.ds(tok0 + j * L, L)])


def kernel(x, norm1_g, norm1_b, qkv_w, qkv_b, rpb_table, proj_w, proj_b,
           norm2_g, norm2_b, gate_w, gate_b, fc1_w, fc1_b, fc2_w, fc2_b,
           rel_pos_index):
    f32 = jnp.float32
    bias = pl.pallas_call(
        _bias_kernel,
        grid=(1,),
        in_specs=[_full((NH, NTAB)), _full((1, N * N))],
        out_specs=_full((NH, N * N)),
        out_shape=jax.ShapeDtypeStruct((NH, N * N), f32),
    )(rpb_table.T, rel_pos_index.reshape(1, N * N))
    bias = bias.reshape(NH, N, N)

    xb = x.reshape(B, NB, WS, W, C)
    r1 = lambda b, r: (b, r, 0, 0, 0)
    x2b, eid = pl.pallas_call(
        _attn_kernel,
        grid=(B, NB),
        in_specs=[
            pl.BlockSpec((1, 1, WS, W, C), r1),
            _full((1, C)), _full((1, C)),
            _full((3 * C, C)), _full((1, 3 * C)),
            _full((NH, N, N)),
            _full((C, C)), _full((1, C)),
            _full((1, C)), _full((1, C)),
            _full((E, C)), _full((1, E)),
        ],
        out_specs=[
            pl.BlockSpec((1, 1, WS, W, C), r1),
            pl.BlockSpec((1, 1, 1, WS * W), lambda b, r: (b, r, 0, 0)),
        ],
        out_shape=[
            jax.ShapeDtypeStruct((B, NB, WS, W, C), f32),
            jax.ShapeDtypeStruct((B, NB, 1, WS * W), jnp.int32),
        ],
        scratch_shapes=[pltpu.VMEM((WS, W, C), f32),
                        pltpu.VMEM(((W // WS) * NH * N, N), f32)],
    )(xb, norm1_g.reshape(1, C), norm1_b.reshape(1, C), qkv_w,
      qkv_b.reshape(1, 3 * C), bias, proj_w, proj_b.reshape(1, C),
      norm2_g.reshape(1, C), norm2_b.reshape(1, C), gate_w,
      gate_b.reshape(1, E))
    x2 = x2b.reshape(TOK, C)
    e_tok = eid.reshape(TOK)

    mesh = plsc.VectorSubcoreMesh(core_axis_name="c", subcore_axis_name="s")
    x_sorted, dst, blk_expert = pl.kernel(
        _route_kernel,
        mesh=mesh,
        compiler_params=pltpu.CompilerParams(needs_layout_passes=False),
        out_type=[
            jax.ShapeDtypeStruct((P, C), f32),
            jax.ShapeDtypeStruct((TOK,), jnp.int32),
            jax.ShapeDtypeStruct((G,), jnp.int32),
        ],
        scratch_types=[
            pltpu.VMEM((TPW,), jnp.int32),
            pltpu.VMEM((L, C), f32),
            pltpu.VMEM((L,), jnp.int32),
            pltpu.VMEM((NS * L,), jnp.int32),
            pltpu.VMEM((L,), jnp.int32),
            pltpu.VMEM((L,), jnp.int32),
            pltpu.VMEM((TPW,), jnp.int32),
            pltpu.VMEM((3 * L,), jnp.int32),
            pltpu.VMEM_SHARED((NS * L,), jnp.int32),
            pltpu.SemaphoreType.DMA,
        ],
    )(e_tok, x2)

    y_sorted = pl.pallas_call(
        _moe_kernel,
        grid_spec=pltpu.PrefetchScalarGridSpec(
            num_scalar_prefetch=1,
            grid=(G,),
            in_specs=[
                pl.BlockSpec((T, C), lambda g, s: (g, 0)),
                _full((1, C)), _full((1, C)),
                pl.BlockSpec((1, HID, C), lambda g, s: (s[g], 0, 0)),
                pl.BlockSpec((1, 1, HID), lambda g, s: (s[g], 0, 0)),
                pl.BlockSpec((1, C, HID), lambda g, s: (s[g], 0, 0)),
                pl.BlockSpec((1, 1, C), lambda g, s: (s[g], 0, 0)),
            ],
            out_specs=pl.BlockSpec((T, C), lambda g, s: (g, 0)),
        ),
        out_shape=jax.ShapeDtypeStruct((P, C), f32),
    )(blk_expert, x_sorted, norm2_g.reshape(1, C), norm2_b.reshape(1, C),
      fc1_w, fc1_b.reshape(E, 1, HID), fc2_w, fc2_b.reshape(E, 1, C))

    out_flat = pl.kernel(
        _unsort_kernel,
        mesh=mesh,
        compiler_params=pltpu.CompilerParams(needs_layout_passes=False),
        out_type=jax.ShapeDtypeStruct((TOK, C), f32),
        scratch_types=[
            pltpu.VMEM((TPW,), jnp.int32),
            pltpu.VMEM((L, C), f32),
            pltpu.SemaphoreType.DMA,
        ],
    )(y_sorted, dst)
    out = out_flat.reshape(B, H * W, C)
    return (out, jnp.float32(0.0))
